# baseline (speedup 1.0000x reference)
import functools

import jax
import jax.numpy as jnp
from jax import lax
from jax.experimental import pallas as pl
from jax.experimental.pallas import tpu as pltpu
from jax.experimental.pallas import tpu_sc as plsc

N_PAD = 10240
C = 256
K = 16
PW = 16
L = 16
NC = 2
NS = 16
CHUNK = 8
NPAIR = N_PAD // CHUNK // NS
N0 = 34
N1 = NPAIR - N0
NMAX = max(N0, N1)


def _table_kernel(f_ref, w_ref, b_ref, o_ref):
    mm = lax.dot_general(f_ref[...], w_ref[...],
                         (((1,), (1,)), ((), ())),
                         preferred_element_type=jnp.float32)
    o_ref[...] = (mm + b_ref[...]).astype(jnp.bfloat16)


def _build_table(feats_p, W, b2):
    BM = 1024
    return pl.pallas_call(
        _table_kernel,
        grid=(N_PAD // BM,),
        in_specs=[
            pl.BlockSpec((BM, C), lambda i: (i, 0)),
            pl.BlockSpec((C, C), lambda i: (0, 0)),
            pl.BlockSpec((1, C), lambda i: (0, 0)),
        ],
        out_specs=pl.BlockSpec((BM, C), lambda i: (i, 0)),
        out_shape=jax.ShapeDtypeStruct((N_PAD, C), jnp.bfloat16),
    )(feats_p, W, b2)


def _resample_body(ftab_hbm, ptab_hbm, idx_hbm, out_hbm, idx_all, self_all,
                   out_all, nbr0, nbr1, pts0, pts1, fsem0, fsem1, psem0,
                   psem1):
    c = lax.axis_index("c")
    s = lax.axis_index("s")
    cbase = s * NPAIR + jnp.where(c == 0, 0, N0)
    base = cbase * CHUNK
    npairs = jnp.where(c == 0, N0 // 2, N1 // 2)
    lanes = lax.broadcasted_iota(jnp.int32, (L,), 0)
    nbr = (nbr0, nbr1)
    pts = (pts0, pts1)
    fsem = (fsem0, fsem1)
    psem = (psem0, psem1)

    pltpu.sync_copy(idx_hbm.at[pl.ds(cbase, NMAX)], idx_all)
    pltpu.sync_copy(ftab_hbm.at[pl.ds(base, NMAX * CHUNK)], self_all)

    def issue(g, b):
        pltpu.async_copy(ftab_hbm.at[idx_all.at[g]], nbr[b], fsem[b])
        pltpu.async_copy(ptab_hbm.at[idx_all.at[g]], pts[b], psem[b])

    issue(0, 0)

    def compute(g, b):
        for i in range(CHUNK):
            p = g * CHUNK + i
            sv = []
            for cb in range(C // 32):
                lo, hi = plsc.unpack(self_all[p, pl.ds(cb * 32, 32)],
                                     format=plsc.PackFormat.INTERLEAVED)
                sv.append(lo)
                sv.append(hi)
            s_log = jnp.zeros((L,), jnp.float32)
            for k in range(K):
                r = i * K + k
                acc = None
                for cb in range(C // 32):
                    lo, hi = plsc.unpack(nbr[b][r, pl.ds(cb * 32, 32)],
                                         format=plsc.PackFormat.INTERLEAVED)
                    t = sv[2 * cb] * lo + sv[2 * cb + 1] * hi
                    acc = t if acc is None else acc + t
                s_log = jnp.where(lanes == k, jnp.sum(acc), s_log)
            s_log = s_log * (1.0 / 16.0)
            m = jnp.max(s_log)
            e = jnp.exp(s_log - m)
            tot = jnp.sum(e)
            ovec = jnp.zeros((L,), jnp.float32)
            for k in range(K):
                e_k = jnp.squeeze(lax.slice_in_dim(e, k, k + 1))
                ovec = ovec + e_k * pts[b][i * K + k, :]
            out_all[p, :] = ovec / jnp.broadcast_to(tot, (L,))

    def pair_body(gp, carry):
        for bb in range(2):
            g = gp * 2 + bb

            @pl.when(g + 1 < carry)
            def _():
                issue(g + 1, 1 - bb)

            pltpu.make_async_copy(
                ftab_hbm.at[idx_all.at[g]], nbr[bb], fsem[bb]).wait()
            pltpu.make_async_copy(
                ptab_hbm.at[idx_all.at[g]], pts[bb], psem[bb]).wait()
            compute(g, bb)
        return carry

    nchunks = npairs * 2
    lax.fori_loop(0, npairs, pair_body, nchunks)

    @pl.when(c == 0)
    def _():
        pltpu.sync_copy(out_all.at[pl.ds(0, N0 * CHUNK)],
                        out_hbm.at[pl.ds(base, N0 * CHUNK)])

    @pl.when(c != 0)
    def _():
        pltpu.sync_copy(out_all.at[pl.ds(0, N1 * CHUNK)],
                        out_hbm.at[pl.ds(base, N1 * CHUNK)])


@functools.cache
def _resample():
    return pl.kernel(
        _resample_body,
        mesh=plsc.VectorSubcoreMesh(core_axis_name="c", subcore_axis_name="s"),
        compiler_params=pltpu.CompilerParams(
            needs_layout_passes=False, use_tc_tiling_on_sc=False),
        out_type=jax.ShapeDtypeStruct((N_PAD, L), jnp.float32),
        scratch_types=[
            pltpu.VMEM((NMAX, CHUNK * K), jnp.int32),
            pltpu.VMEM((NMAX * CHUNK, C), jnp.bfloat16),
            pltpu.VMEM((NMAX * CHUNK, L), jnp.float32),
            pltpu.VMEM((CHUNK * K, C), jnp.bfloat16),
            pltpu.VMEM((CHUNK * K, C), jnp.bfloat16),
            pltpu.VMEM((CHUNK * K, PW), jnp.float32),
            pltpu.VMEM((CHUNK * K, PW), jnp.float32),
            pltpu.SemaphoreType.DMA,
            pltpu.SemaphoreType.DMA,
            pltpu.SemaphoreType.DMA,
            pltpu.SemaphoreType.DMA,
        ],
    )


def kernel(points, feats, neighbor_indices, W, b):
    n, k = neighbor_indices.shape
    rows = jnp.arange(n, dtype=neighbor_indices.dtype)[:, None]
    idx = jnp.where(neighbor_indices < n, neighbor_indices,
                    jnp.broadcast_to(rows, (n, k))).astype(jnp.int32)
    feats_p = jnp.pad(feats, ((0, N_PAD - n), (0, 0)))
    ptab = jnp.pad(points, ((0, N_PAD - n), (0, PW - 3)))
    idx2 = jnp.pad(idx, ((0, N_PAD - n), (0, 0))).reshape(-1, CHUNK * K)
    ftab = _build_table(feats_p, W, b.reshape(1, C))
    out = _resample()(ftab, ptab, idx2)
    return out[:n, :3]


# speedup vs baseline: 1.1111x; 1.1111x over previous
import functools

import jax
import jax.numpy as jnp
from jax import lax
from jax.experimental import pallas as pl
from jax.experimental.pallas import tpu as pltpu
from jax.experimental.pallas import tpu_sc as plsc

N_PAD = 10240
C = 256
K = 16
PW = 16
L = 16
NC = 2
NS = 16
CHUNK = 8
NPAIR = N_PAD // CHUNK // NS
N0 = 46
N1 = NPAIR - N0
NMAX = max(N0, N1)


def _table_kernel(f_ref, w_ref, b_ref, o_ref):
    mm = lax.dot_general(f_ref[...], w_ref[...],
                         (((1,), (1,)), ((), ())),
                         preferred_element_type=jnp.float32)
    o_ref[...] = (mm + b_ref[...]).astype(jnp.bfloat16)


def _build_table(feats_p, W, b2):
    BM = 1024
    return pl.pallas_call(
        _table_kernel,
        grid=(N_PAD // BM,),
        in_specs=[
            pl.BlockSpec((BM, C), lambda i: (i, 0)),
            pl.BlockSpec((C, C), lambda i: (0, 0)),
            pl.BlockSpec((1, C), lambda i: (0, 0)),
        ],
        out_specs=pl.BlockSpec((BM, C), lambda i: (i, 0)),
        out_shape=jax.ShapeDtypeStruct((N_PAD, C), jnp.bfloat16),
    )(feats_p, W, b2)


def _resample_body(ftab_hbm, ptab_hbm, idx_hbm, out_hbm, idx_all, self_all,
                   out_all, nbr0, nbr1, pts0, pts1, fsem0, fsem1, psem0,
                   psem1):
    c = lax.axis_index("c")
    s = lax.axis_index("s")
    cbase = s * NPAIR + jnp.where(c == 0, 0, N0)
    base = cbase * CHUNK
    npairs = jnp.where(c == 0, N0 // 2, N1 // 2)
    lanes = lax.broadcasted_iota(jnp.int32, (L,), 0)
    nbr = (nbr0, nbr1)
    pts = (pts0, pts1)
    fsem = (fsem0, fsem1)
    psem = (psem0, psem1)

    pltpu.sync_copy(idx_hbm.at[pl.ds(cbase, NMAX)], idx_all)
    pltpu.sync_copy(ftab_hbm.at[pl.ds(base, NMAX * CHUNK)], self_all)

    def issue(g, b):
        pltpu.async_copy(ftab_hbm.at[idx_all.at[g]], nbr[b], fsem[b])
        pltpu.async_copy(ptab_hbm.at[idx_all.at[g]], pts[b], psem[b])

    issue(0, 0)

    def compute(g, b):
        for i in range(CHUNK):
            p = g * CHUNK + i
            sv = []
            for cb in range(C // 32):
                lo, hi = plsc.unpack(self_all[p, pl.ds(cb * 32, 32)],
                                     format=plsc.PackFormat.INTERLEAVED)
                sv.append(lo)
                sv.append(hi)
            s_log = jnp.zeros((L,), jnp.float32)
            for k in range(K):
                r = i * K + k
                acc = None
                for cb in range(C // 32):
                    lo, hi = plsc.unpack(nbr[b][r, pl.ds(cb * 32, 32)],
                                         format=plsc.PackFormat.INTERLEAVED)
                    t = sv[2 * cb] * lo + sv[2 * cb + 1] * hi
                    acc = t if acc is None else acc + t
                s_log = jnp.where(lanes == k, jnp.sum(acc), s_log)
            s_log = s_log * (1.0 / 16.0)
            m = jnp.max(s_log)
            e = jnp.exp(s_log - m)
            tot = jnp.sum(e)
            ovec = jnp.zeros((L,), jnp.float32)
            for k in range(K):
                e_k = jnp.squeeze(lax.slice_in_dim(e, k, k + 1))
                ovec = ovec + e_k * pts[b][i * K + k, :]
            out_all[p, :] = ovec / jnp.broadcast_to(tot, (L,))

    def pair_body(gp, carry):
        for bb in range(2):
            g = gp * 2 + bb

            @pl.when(g + 1 < carry)
            def _():
                issue(g + 1, 1 - bb)

            pltpu.make_async_copy(
                ftab_hbm.at[idx_all.at[g]], nbr[bb], fsem[bb]).wait()
            pltpu.make_async_copy(
                ptab_hbm.at[idx_all.at[g]], pts[bb], psem[bb]).wait()
            compute(g, bb)
        return carry

    nchunks = npairs * 2
    lax.fori_loop(0, npairs, pair_body, nchunks)

    @pl.when(c == 0)
    def _():
        pltpu.sync_copy(out_all.at[pl.ds(0, N0 * CHUNK)],
                        out_hbm.at[pl.ds(base, N0 * CHUNK)])

    @pl.when(c != 0)
    def _():
        pltpu.sync_copy(out_all.at[pl.ds(0, N1 * CHUNK)],
                        out_hbm.at[pl.ds(base, N1 * CHUNK)])


@functools.cache
def _resample():
    return pl.kernel(
        _resample_body,
        mesh=plsc.VectorSubcoreMesh(core_axis_name="c", subcore_axis_name="s"),
        compiler_params=pltpu.CompilerParams(
            needs_layout_passes=False, use_tc_tiling_on_sc=False),
        out_type=jax.ShapeDtypeStruct((N_PAD, L), jnp.float32),
        scratch_types=[
            pltpu.VMEM((NMAX, CHUNK * K), jnp.int32),
            pltpu.VMEM((NMAX * CHUNK, C), jnp.bfloat16),
            pltpu.VMEM((NMAX * CHUNK, L), jnp.float32),
            pltpu.VMEM((CHUNK * K, C), jnp.bfloat16),
            pltpu.VMEM((CHUNK * K, C), jnp.bfloat16),
            pltpu.VMEM((CHUNK * K, PW), jnp.float32),
            pltpu.VMEM((CHUNK * K, PW), jnp.float32),
            pltpu.SemaphoreType.DMA,
            pltpu.SemaphoreType.DMA,
            pltpu.SemaphoreType.DMA,
            pltpu.SemaphoreType.DMA,
        ],
    )


def kernel(points, feats, neighbor_indices, W, b):
    n, k = neighbor_indices.shape
    rows = jnp.arange(n, dtype=neighbor_indices.dtype)[:, None]
    idx = jnp.where(neighbor_indices < n, neighbor_indices,
                    jnp.broadcast_to(rows, (n, k))).astype(jnp.int32)
    feats_p = jnp.pad(feats, ((0, N_PAD - n), (0, 0)))
    ptab = jnp.pad(points, ((0, N_PAD - n), (0, PW - 3)))
    idx2 = jnp.pad(idx, ((0, N_PAD - n), (0, 0))).reshape(-1, CHUNK * K)
    ftab = _build_table(feats_p, W, b.reshape(1, C))
    out = _resample()(ftab, ptab, idx2)
    return out[:n, :3]


# speedup vs baseline: 1.1362x; 1.0226x over previous
import functools

import jax
import jax.numpy as jnp
from jax import lax
from jax.experimental import pallas as pl
from jax.experimental.pallas import tpu as pltpu
from jax.experimental.pallas import tpu_sc as plsc

N_PAD = 10240
C = 256
K = 16
PW = 16
L = 16
NC = 2
NS = 16
CHUNK = 8
NPAIR = N_PAD // CHUNK // NS
N0 = 48
N1 = NPAIR - N0
NMAX = max(N0, N1)


def _table_kernel(f_ref, w_ref, b_ref, o_ref):
    mm = lax.dot_general(f_ref[...], w_ref[...],
                         (((1,), (1,)), ((), ())),
                         preferred_element_type=jnp.float32)
    o_ref[...] = (mm + b_ref[...]).astype(jnp.bfloat16)


def _build_table(feats_p, W, b2):
    BM = 1024
    return pl.pallas_call(
        _table_kernel,
        grid=(N_PAD // BM,),
        in_specs=[
            pl.BlockSpec((BM, C), lambda i: (i, 0)),
            pl.BlockSpec((C, C), lambda i: (0, 0)),
            pl.BlockSpec((1, C), lambda i: (0, 0)),
        ],
        out_specs=pl.BlockSpec((BM, C), lambda i: (i, 0)),
        out_shape=jax.ShapeDtypeStruct((N_PAD, C), jnp.bfloat16),
    )(feats_p, W, b2)


def _resample_body(ftab_hbm, ptab_hbm, idx_hbm, out_hbm, idx_all, self_all,
                   out_all, nbr0, nbr1, pts0, pts1, fsem0, fsem1, psem0,
                   psem1):
    c = lax.axis_index("c")
    s = lax.axis_index("s")
    cbase = s * NPAIR + jnp.where(c == 0, 0, N0)
    base = cbase * CHUNK
    npairs = jnp.where(c == 0, N0 // 2, N1 // 2)
    lanes = lax.broadcasted_iota(jnp.int32, (L,), 0)
    nbr = (nbr0, nbr1)
    pts = (pts0, pts1)
    fsem = (fsem0, fsem1)
    psem = (psem0, psem1)

    pltpu.sync_copy(idx_hbm.at[pl.ds(cbase, NMAX)], idx_all)
    pltpu.sync_copy(ftab_hbm.at[pl.ds(base, NMAX * CHUNK)], self_all)

    def issue(g, b):
        pltpu.async_copy(ftab_hbm.at[idx_all.at[g]], nbr[b], fsem[b])
        pltpu.async_copy(ptab_hbm.at[idx_all.at[g]], pts[b], psem[b])

    issue(0, 0)

    def compute(g, b):
        for i in range(CHUNK):
            p = g * CHUNK + i
            sv = [self_all[p, pl.ds(cb * 32, 32)] for cb in range(C // 32)]
            s_log = jnp.zeros((L,), jnp.float32)
            for k in range(K):
                r = i * K + k
                acc = None
                for cb in range(C // 32):
                    prod = sv[cb] * nbr[b][r, pl.ds(cb * 32, 32)]
                    lo, hi = plsc.unpack(prod,
                                         format=plsc.PackFormat.INTERLEAVED)
                    t = lo + hi
                    acc = t if acc is None else acc + t
                s_log = jnp.where(lanes == k, jnp.sum(acc), s_log)
            s_log = s_log * (1.0 / 16.0)
            m = jnp.max(s_log)
            e = jnp.exp(s_log - m)
            tot = jnp.sum(e)
            ovec = jnp.zeros((L,), jnp.float32)
            for k in range(K):
                e_k = jnp.squeeze(lax.slice_in_dim(e, k, k + 1))
                ovec = ovec + e_k * pts[b][i * K + k, :]
            out_all[p, :] = ovec / jnp.broadcast_to(tot, (L,))

    def pair_body(gp, carry):
        for bb in range(2):
            g = gp * 2 + bb

            @pl.when(g + 1 < carry)
            def _():
                issue(g + 1, 1 - bb)

            pltpu.make_async_copy(
                ftab_hbm.at[idx_all.at[g]], nbr[bb], fsem[bb]).wait()
            pltpu.make_async_copy(
                ptab_hbm.at[idx_all.at[g]], pts[bb], psem[bb]).wait()
            compute(g, bb)
        return carry

    nchunks = npairs * 2
    lax.fori_loop(0, npairs, pair_body, nchunks)

    @pl.when(c == 0)
    def _():
        pltpu.sync_copy(out_all.at[pl.ds(0, N0 * CHUNK)],
                        out_hbm.at[pl.ds(base, N0 * CHUNK)])

    @pl.when(c != 0)
    def _():
        pltpu.sync_copy(out_all.at[pl.ds(0, N1 * CHUNK)],
                        out_hbm.at[pl.ds(base, N1 * CHUNK)])


@functools.cache
def _resample():
    return pl.kernel(
        _resample_body,
        mesh=plsc.VectorSubcoreMesh(core_axis_name="c", subcore_axis_name="s"),
        compiler_params=pltpu.CompilerParams(
            needs_layout_passes=False, use_tc_tiling_on_sc=False),
        out_type=jax.ShapeDtypeStruct((N_PAD, L), jnp.float32),
        scratch_types=[
            pltpu.VMEM((NMAX, CHUNK * K), jnp.int32),
            pltpu.VMEM((NMAX * CHUNK, C), jnp.bfloat16),
            pltpu.VMEM((NMAX * CHUNK, L), jnp.float32),
            pltpu.VMEM((CHUNK * K, C), jnp.bfloat16),
            pltpu.VMEM((CHUNK * K, C), jnp.bfloat16),
            pltpu.VMEM((CHUNK * K, PW), jnp.float32),
            pltpu.VMEM((CHUNK * K, PW), jnp.float32),
            pltpu.SemaphoreType.DMA,
            pltpu.SemaphoreType.DMA,
            pltpu.SemaphoreType.DMA,
            pltpu.SemaphoreType.DMA,
        ],
    )


def kernel(points, feats, neighbor_indices, W, b):
    n, k = neighbor_indices.shape
    rows = jnp.arange(n, dtype=neighbor_indices.dtype)[:, None]
    idx = jnp.where(neighbor_indices < n, neighbor_indices,
                    jnp.broadcast_to(rows, (n, k))).astype(jnp.int32)
    feats_p = jnp.pad(feats, ((0, N_PAD - n), (0, 0)))
    ptab = jnp.pad(points, ((0, N_PAD - n), (0, PW - 3)))
    idx2 = jnp.pad(idx, ((0, N_PAD - n), (0, 0))).reshape(-1, CHUNK * K)
    ftab = _build_table(feats_p, W, b.reshape(1, C))
    out = _resample()(ftab, ptab, idx2)
    return out[:n, :3]
